# baseline (speedup 1.0000x reference)
import functools

import jax
import jax.numpy as jnp
from jax import lax
from jax.experimental import pallas as pl
from jax.experimental.pallas import tpu as pltpu
from jax.experimental.pallas import tpu_sc as plsc

N = 10000
E = 320000
EL = 100000
D = 128

NC = 2
NS = 16
NW = NC * NS
L = 16

NPAD = 10240
RPS = NPAD // NS
DUMMY = N

EB = 128
TOTB = 2560
EPAD = TOTB * EB
NB0 = 120
NB1 = 40
CHB = 8
NCH0 = NB0 // CHB
NCH1 = NB1 // CHB
NB_DEG = TOTB // NW

TOTBL = 896
ELPAD = TOTBL * 128
NBL0 = 48
NBL1 = 8


def _mesh():
    return plsc.VectorSubcoreMesh(
        core_axis_name="c", subcore_axis_name="s", num_cores=NC, num_subcores=NS
    )


def _deg_call(dst2, z16):
    @functools.partial(
        pl.kernel,
        out_type=jax.ShapeDtypeStruct((NC, NPAD, L), jnp.float32),
        mesh=_mesh(),
        scratch_types=[
            pltpu.VMEM((NB_DEG, EB), jnp.int32),
            pltpu.VMEM((EB, L), jnp.float32),
            pltpu.VMEM_SHARED((NPAD, L), jnp.float32),
            pltpu.SemaphoreType.DMA,
        ],
        name="sc_deg",
    )
    def deg_kernel(dst_hbm, z_hbm, out_hbm, dstv, onesv, acc, semz):
        cid = lax.axis_index("c")
        sid = lax.axis_index("s")
        wid = sid * NC + cid

        zc = pltpu.async_copy(
            z_hbm.at[pl.ds(sid * RPS, RPS)], acc.at[pl.ds(sid * RPS, RPS)], semz
        )
        pltpu.sync_copy(dst_hbm.at[pl.ds(wid * NB_DEG, NB_DEG)], dstv)

        @pl.loop(0, EB)
        def _(i):
            onesv[pl.ds(i, 1), :] = jnp.ones((1, L), jnp.float32)

        zc.wait()
        plsc.subcore_barrier()

        @pl.loop(0, NB_DEG)
        def _(j):
            pltpu.sync_copy(onesv, acc.at[dstv.at[j]], add=True)

        plsc.subcore_barrier()
        pltpu.sync_copy(
            acc.at[pl.ds(sid * RPS, RPS)], out_hbm.at[cid].at[pl.ds(sid * RPS, RPS)]
        )

    return deg_kernel(dst2, z16)


def _prop_call(xp, src2, dst2, z128):
    @functools.partial(
        pl.kernel,
        out_type=jax.ShapeDtypeStruct((NC, NPAD, D), jnp.float32),
        mesh=_mesh(),
        scratch_types=[
            pltpu.VMEM((2 * CHB, EB), jnp.int32),
            pltpu.VMEM((2 * CHB, EB), jnp.int32),
            pltpu.VMEM((EB, D), jnp.float32),
            pltpu.VMEM((EB, D), jnp.float32),
            pltpu.VMEM_SHARED((NPAD, D), jnp.float32),
            pltpu.SemaphoreType.DMA,
            pltpu.SemaphoreType.DMA,
            pltpu.SemaphoreType.DMA,
            pltpu.SemaphoreType.DMA,
            pltpu.SemaphoreType.DMA,
        ],
        name="sc_prop",
    )
    def prop_kernel(
        xp_hbm, src_hbm, dst_hbm, z_hbm, out_hbm,
        srcv, dstv, bufa, bufb, acc, semz, sema, semb, semis, semid,
    ):
        cid = lax.axis_index("c")
        sid = lax.axis_index("s")

        nch = jnp.where(cid == 0, NCH0, NCH1)
        wb = jnp.where(cid == 0, sid * NB0, NS * NB0 + sid * NB1)

        zc = pltpu.async_copy(
            z_hbm.at[pl.ds(sid * RPS, RPS)], acc.at[pl.ds(sid * RPS, RPS)], semz
        )
        pltpu.sync_copy(src_hbm.at[pl.ds(wb, CHB)], srcv.at[pl.ds(0, CHB)])
        pltpu.sync_copy(dst_hbm.at[pl.ds(wb, CHB)], dstv.at[pl.ds(0, CHB)])
        zc.wait()
        plsc.subcore_barrier()

        pltpu.async_copy(xp_hbm.at[srcv.at[0]], bufa, sema)

        @pl.loop(0, nch)
        def _(c):
            par = lax.rem(c, 2)
            base = par * CHB
            nbase = (1 - par) * CHB

            @pl.when(c + 1 < nch)
            def _():
                pltpu.async_copy(
                    src_hbm.at[pl.ds(wb + (c + 1) * CHB, CHB)],
                    srcv.at[pl.ds(nbase, CHB)], semis,
                )
                pltpu.async_copy(
                    dst_hbm.at[pl.ds(wb + (c + 1) * CHB, CHB)],
                    dstv.at[pl.ds(nbase, CHB)], semid,
                )

            @pl.loop(0, CHB, step=2)
            def _(jj):
                pltpu.make_async_copy(
                    xp_hbm.at[srcv.at[base + jj]], bufa, sema
                ).wait()
                pltpu.async_copy(xp_hbm.at[srcv.at[base + jj + 1]], bufb, semb)
                pltpu.sync_copy(bufa, acc.at[dstv.at[base + jj]], add=True)
                pltpu.make_async_copy(
                    xp_hbm.at[srcv.at[base + jj + 1]], bufb, semb
                ).wait()

                @pl.when(jj + 2 < CHB)
                def _():
                    pltpu.async_copy(
                        xp_hbm.at[srcv.at[base + jj + 2]], bufa, sema
                    )

                pltpu.sync_copy(bufb, acc.at[dstv.at[base + jj + 1]], add=True)

            @pl.when(c + 1 < nch)
            def _():
                pltpu.make_async_copy(
                    src_hbm.at[pl.ds(wb + (c + 1) * CHB, CHB)],
                    srcv.at[pl.ds(nbase, CHB)], semis,
                ).wait()
                pltpu.make_async_copy(
                    dst_hbm.at[pl.ds(wb + (c + 1) * CHB, CHB)],
                    dstv.at[pl.ds(nbase, CHB)], semid,
                ).wait()
                pltpu.async_copy(xp_hbm.at[srcv.at[nbase]], bufa, sema)

        plsc.subcore_barrier()
        pltpu.sync_copy(
            acc.at[pl.ds(sid * RPS, RPS)], out_hbm.at[cid].at[pl.ds(sid * RPS, RPS)]
        )

    return prop_kernel(xp, src2, dst2, z128)


def _decode_call(agg2, a0, a1, b0, b1):
    @functools.partial(
        pl.kernel,
        out_type=[
            jax.ShapeDtypeStruct((NS * NBL0 * 16, 128), jnp.float32),
            jax.ShapeDtypeStruct((NS * NBL1 * 16, 128), jnp.float32),
        ],
        mesh=_mesh(),
        scratch_types=[
            pltpu.VMEM((NBL0, 128), jnp.int32),
            pltpu.VMEM((NBL0, 128), jnp.int32),
            pltpu.VMEM((NBL1, 128), jnp.int32),
            pltpu.VMEM((NBL1, 128), jnp.int32),
            pltpu.VMEM((128, D), jnp.float32),
            pltpu.VMEM((128, D), jnp.float32),
            pltpu.VMEM((2 * 16, 128), jnp.float32),
            pltpu.SemaphoreType.DMA,
            pltpu.SemaphoreType.DMA,
            pltpu.SemaphoreType.DMA,
        ],
        name="sc_decode",
    )
    def dec_kernel(emb_hbm, a0_hbm, a1_hbm, b0_hbm, b1_hbm, d0_hbm, d1_hbm,
                   av0, bv0, av1, bv1, bufa, bufb, outv, sema, semb, semo):
        cid = lax.axis_index("c")
        sid = lax.axis_index("s")

        def work(avx, bvx, nblx, dx_hbm):
            obase = sid * (nblx * 16)

            @pl.loop(0, nblx)
            def _(j):
                ca = pltpu.async_copy(emb_hbm.at[avx.at[j]], bufa, sema)
                cb = pltpu.async_copy(emb_hbm.at[bvx.at[j]], bufb, semb)
                ca.wait()
                cb.wait()
                par16 = lax.rem(j, 2) * 16

                @pl.when(j >= 2)
                def _():
                    pltpu.make_async_copy(
                        outv.at[pl.ds(par16, 16)],
                        dx_hbm.at[pl.ds(obase, 16)], semo,
                    ).wait()

                @pl.loop(0, 128)
                def _(i):
                    acc = (
                        bufa[pl.ds(i, 1), pl.ds(0, L)]
                        * bufb[pl.ds(i, 1), pl.ds(0, L)]
                    )
                    for c in range(1, D // L):
                        acc = acc + (
                            bufa[pl.ds(i, 1), pl.ds(c * L, L)]
                            * bufb[pl.ds(i, 1), pl.ds(c * L, L)]
                        )
                    outv[pl.ds(par16 + i // 8, 1), pl.ds((i % 8) * L, L)] = acc

                pltpu.async_copy(
                    outv.at[pl.ds(par16, 16)],
                    dx_hbm.at[pl.ds(obase + j * 16, 16)], semo,
                )

            @pl.loop(0, 2)
            def _(k):
                pltpu.make_async_copy(
                    outv.at[pl.ds(0, 16)], dx_hbm.at[pl.ds(obase, 16)], semo
                ).wait()

        @pl.when(cid == 0)
        def _():
            pltpu.sync_copy(a0_hbm.at[sid], av0)
            pltpu.sync_copy(b0_hbm.at[sid], bv0)
            work(av0, bv0, NBL0, d0_hbm)

        @pl.when(cid == 1)
        def _():
            pltpu.sync_copy(a1_hbm.at[sid], av1)
            pltpu.sync_copy(b1_hbm.at[sid], bv1)
            work(av1, bv1, NBL1, d1_hbm)

    return dec_kernel(agg2, a0, a1, b0, b1)


_DGRID = 4
_DROWS = ELPAD // 128 // _DGRID


def _decode_finish_call(dots3):
    def body(d_ref, o_ref):
        o_ref[...] = jax.nn.sigmoid(jnp.sum(d_ref[...], axis=-1))

    return pl.pallas_call(
        body,
        grid=(_DGRID,),
        in_specs=[pl.BlockSpec((_DROWS, 128, L), lambda i: (i, 0, 0))],
        out_specs=pl.BlockSpec((_DROWS, 128), lambda i: (i, 0)),
        out_shape=jax.ShapeDtypeStruct((ELPAD // 128, 128), jnp.float32),
    )(dots3)


_GRID = 4
_BLK = NPAD // _GRID


def _row_spec(w):
    return pl.BlockSpec((_BLK, w), lambda i: (i, 0))


def _full_spec(h, w):
    return pl.BlockSpec((h, w), lambda i: (0, 0))


def _dinv(dga_ref, dgb_ref):
    deg = dga_ref[:, :1] + dgb_ref[:, :1] + 1.0
    return lax.rsqrt(deg)


def _xp_call(dga, dgb, x_pad):
    def body(dga_ref, dgb_ref, x_ref, o_ref):
        o_ref[...] = x_ref[...] * _dinv(dga_ref, dgb_ref)

    return pl.pallas_call(
        body,
        grid=(_GRID,),
        in_specs=[_row_spec(L), _row_spec(L), _row_spec(D)],
        out_specs=_row_spec(D),
        out_shape=jax.ShapeDtypeStruct((NPAD, D), jnp.float32),
    )(dga, dgb, x_pad)


def _layer1_call(p0, p1, xp, dga, dgb, W1, b1):
    def body(p0_ref, p1_ref, xp_ref, dga_ref, dgb_ref, w_ref, b_ref, o_ref):
        dinv = _dinv(dga_ref, dgb_ref)
        agg = (p0_ref[...] + p1_ref[...] + xp_ref[...]) * dinv
        h = jnp.dot(agg, w_ref[...], precision=lax.Precision.HIGHEST) + b_ref[...]
        o_ref[...] = jnp.maximum(h, 0.0) * dinv

    return pl.pallas_call(
        body,
        grid=(_GRID,),
        in_specs=[
            _row_spec(D), _row_spec(D), _row_spec(D),
            _row_spec(L), _row_spec(L),
            _full_spec(D, D), _full_spec(1, D),
        ],
        out_specs=_row_spec(D),
        out_shape=jax.ShapeDtypeStruct((NPAD, D), jnp.float32),
    )(p0, p1, xp, dga, dgb, W1, b1)


def _layer2_call(q0, q1, h1p, dga, dgb, W2, b2):
    def body(q0_ref, q1_ref, h1p_ref, dga_ref, dgb_ref, w_ref, b_ref,
             agg_ref, h_ref):
        dinv = _dinv(dga_ref, dgb_ref)
        agg = (q0_ref[...] + q1_ref[...] + h1p_ref[...]) * dinv
        agg_ref[...] = agg
        h_ref[...] = (
            jnp.dot(agg, w_ref[...], precision=lax.Precision.HIGHEST) + b_ref[...]
        )

    return pl.pallas_call(
        body,
        grid=(_GRID,),
        in_specs=[
            _row_spec(D), _row_spec(D), _row_spec(D),
            _row_spec(L), _row_spec(L),
            _full_spec(D, D), _full_spec(1, D),
        ],
        out_specs=[_row_spec(D), _row_spec(D)],
        out_shape=[
            jax.ShapeDtypeStruct((NPAD, D), jnp.float32),
            jax.ShapeDtypeStruct((NPAD, D), jnp.float32),
        ],
    )(q0, q1, h1p, dga, dgb, W2, b2)


def _pad_idx(idx, total, fill):
    pad = jnp.full((total - idx.shape[0],), fill, jnp.int32)
    return jnp.concatenate([idx.astype(jnp.int32), pad])


def kernel(x, edge_index, edge_label_index, W1, b1, W2, b2):
    src2 = _pad_idx(edge_index[0], EPAD, DUMMY).reshape(TOTB, EB)
    dst2 = _pad_idx(edge_index[1], EPAD, DUMMY).reshape(TOTB, EB)
    a2 = _pad_idx(edge_label_index[0], ELPAD, 0).reshape(TOTBL, 128)
    b2_idx = _pad_idx(edge_label_index[1], ELPAD, 0).reshape(TOTBL, 128)
    split = NS * NBL0
    ai0 = a2[:split].reshape(NS, NBL0, 128)
    ai1 = a2[split:].reshape(NS, NBL1, 128)
    bi0 = b2_idx[:split].reshape(NS, NBL0, 128)
    bi1 = b2_idx[split:].reshape(NS, NBL1, 128)

    x_pad = jnp.pad(x, ((0, NPAD - N), (0, 0)))
    z16 = jnp.zeros((NPAD, L), jnp.float32)
    z128 = jnp.zeros((NPAD, D), jnp.float32)

    degp = _deg_call(dst2, z16)
    dga, dgb = degp[0], degp[1]

    xp = _xp_call(dga, dgb, x_pad)
    p = _prop_call(xp, src2, dst2, z128)
    h1p = _layer1_call(p[0], p[1], xp, dga, dgb, W1, b1.reshape(1, D))
    q = _prop_call(h1p, src2, dst2, z128)
    agg2, h2 = _layer2_call(q[0], q[1], h1p, dga, dgb, W2, b2.reshape(1, D))
    d0, d1 = _decode_call(agg2, ai0, ai1, bi0, bi1)
    dots = jnp.concatenate([d0, d1], axis=0)
    r = _decode_finish_call(dots.reshape(ELPAD // 128, 128, L)).reshape(ELPAD)

    return (h2[:N], r[:EL])


# speedup vs baseline: 3.0109x; 3.0109x over previous
import functools

import jax
import jax.numpy as jnp
from jax import lax
from jax.experimental import pallas as pl
from jax.experimental.pallas import tpu as pltpu
from jax.experimental.pallas import tpu_sc as plsc

N = 10000
E = 320000
EL = 100000
D = 128

NC = 2
NS = 16
NW = NC * NS
L = 16

NPAD = 10240
RPS = NPAD // NS
DUMMY = N

EB = 128
TOTB = 2560
EPAD = TOTB * EB
NB0 = 120
NB1 = 40
CHB = 8
NCH0 = NB0 // CHB
NCH1 = NB1 // CHB
NB_DEG = TOTB // NW

TOTBL = 896
ELPAD = TOTBL * 128
NBL0 = 48
NBL1 = 8


def _mesh():
    return plsc.VectorSubcoreMesh(
        core_axis_name="c", subcore_axis_name="s", num_cores=NC, num_subcores=NS
    )


def _deg_call(dst2, z16):
    @functools.partial(
        pl.kernel,
        out_type=jax.ShapeDtypeStruct((NC, NPAD, L), jnp.float32),
        mesh=_mesh(),
        scratch_types=[
            pltpu.VMEM((NB_DEG, EB), jnp.int32),
            pltpu.VMEM((EB, L), jnp.float32),
            pltpu.VMEM_SHARED((NPAD, L), jnp.float32),
            pltpu.SemaphoreType.DMA,
        ],
        name="sc_deg",
    )
    def deg_kernel(dst_hbm, z_hbm, out_hbm, dstv, onesv, acc, semz):
        cid = lax.axis_index("c")
        sid = lax.axis_index("s")
        wid = sid * NC + cid

        zc = pltpu.async_copy(
            z_hbm.at[pl.ds(sid * RPS, RPS)], acc.at[pl.ds(sid * RPS, RPS)], semz
        )
        pltpu.sync_copy(dst_hbm.at[pl.ds(wid * NB_DEG, NB_DEG)], dstv)

        @pl.loop(0, EB)
        def _(i):
            onesv[pl.ds(i, 1), :] = jnp.ones((1, L), jnp.float32)

        zc.wait()
        plsc.subcore_barrier()

        @pl.loop(0, NB_DEG)
        def _(j):
            pltpu.sync_copy(onesv, acc.at[dstv.at[j]], add=True)

        plsc.subcore_barrier()
        pltpu.sync_copy(
            acc.at[pl.ds(sid * RPS, RPS)], out_hbm.at[cid].at[pl.ds(sid * RPS, RPS)]
        )

    return deg_kernel(dst2, z16)


def _prop_call(xp, src2, dst2, z128):
    @functools.partial(
        pl.kernel,
        out_type=jax.ShapeDtypeStruct((NC, NPAD, D), jnp.float32),
        mesh=_mesh(),
        scratch_types=[
            pltpu.VMEM((2 * CHB, EB), jnp.int32),
            pltpu.VMEM((2 * CHB, EB), jnp.int32),
            pltpu.VMEM((EB, D), jnp.float32),
            pltpu.VMEM((EB, D), jnp.float32),
            pltpu.VMEM_SHARED((NPAD, D), jnp.float32),
            pltpu.SemaphoreType.DMA,
            pltpu.SemaphoreType.DMA,
            pltpu.SemaphoreType.DMA,
            pltpu.SemaphoreType.DMA,
            pltpu.SemaphoreType.DMA,
        ],
        name="sc_prop",
    )
    def prop_kernel(
        xp_hbm, src_hbm, dst_hbm, z_hbm, out_hbm,
        srcv, dstv, bufa, bufb, acc, semz, sema, semb, semis, semid,
    ):
        cid = lax.axis_index("c")
        sid = lax.axis_index("s")

        nch = jnp.where(cid == 0, NCH0, NCH1)
        wb = jnp.where(cid == 0, sid * NB0, NS * NB0 + sid * NB1)

        zc = pltpu.async_copy(
            z_hbm.at[pl.ds(sid * RPS, RPS)], acc.at[pl.ds(sid * RPS, RPS)], semz
        )
        pltpu.sync_copy(src_hbm.at[pl.ds(wb, CHB)], srcv.at[pl.ds(0, CHB)])
        pltpu.sync_copy(dst_hbm.at[pl.ds(wb, CHB)], dstv.at[pl.ds(0, CHB)])
        zc.wait()
        plsc.subcore_barrier()

        pltpu.async_copy(xp_hbm.at[srcv.at[0]], bufa, sema)

        @pl.loop(0, nch)
        def _(c):
            par = lax.rem(c, 2)
            base = par * CHB
            nbase = (1 - par) * CHB

            @pl.when(c + 1 < nch)
            def _():
                pltpu.async_copy(
                    src_hbm.at[pl.ds(wb + (c + 1) * CHB, CHB)],
                    srcv.at[pl.ds(nbase, CHB)], semis,
                )
                pltpu.async_copy(
                    dst_hbm.at[pl.ds(wb + (c + 1) * CHB, CHB)],
                    dstv.at[pl.ds(nbase, CHB)], semid,
                )

            @pl.loop(0, CHB, step=2)
            def _(jj):
                pltpu.make_async_copy(
                    xp_hbm.at[srcv.at[base + jj]], bufa, sema
                ).wait()
                pltpu.async_copy(xp_hbm.at[srcv.at[base + jj + 1]], bufb, semb)
                pltpu.sync_copy(bufa, acc.at[dstv.at[base + jj]], add=True)
                pltpu.make_async_copy(
                    xp_hbm.at[srcv.at[base + jj + 1]], bufb, semb
                ).wait()

                @pl.when(jj + 2 < CHB)
                def _():
                    pltpu.async_copy(
                        xp_hbm.at[srcv.at[base + jj + 2]], bufa, sema
                    )

                pltpu.sync_copy(bufb, acc.at[dstv.at[base + jj + 1]], add=True)

            @pl.when(c + 1 < nch)
            def _():
                pltpu.make_async_copy(
                    src_hbm.at[pl.ds(wb + (c + 1) * CHB, CHB)],
                    srcv.at[pl.ds(nbase, CHB)], semis,
                ).wait()
                pltpu.make_async_copy(
                    dst_hbm.at[pl.ds(wb + (c + 1) * CHB, CHB)],
                    dstv.at[pl.ds(nbase, CHB)], semid,
                ).wait()
                pltpu.async_copy(xp_hbm.at[srcv.at[nbase]], bufa, sema)

        plsc.subcore_barrier()
        pltpu.sync_copy(
            acc.at[pl.ds(sid * RPS, RPS)], out_hbm.at[cid].at[pl.ds(sid * RPS, RPS)]
        )

    return prop_kernel(xp, src2, dst2, z128)


def _decode_call(agg2, a0, a1, b0, b1):
    @functools.partial(
        pl.kernel,
        out_type=[
            jax.ShapeDtypeStruct((NS * NBL0 * 16, 128), jnp.float32),
            jax.ShapeDtypeStruct((NS * NBL1 * 16, 128), jnp.float32),
        ],
        mesh=_mesh(),
        scratch_types=[
            pltpu.VMEM((NBL0, 128), jnp.int32),
            pltpu.VMEM((NBL0, 128), jnp.int32),
            pltpu.VMEM((NBL1, 128), jnp.int32),
            pltpu.VMEM((NBL1, 128), jnp.int32),
            pltpu.VMEM((128, D), jnp.float32),
            pltpu.VMEM((128, D), jnp.float32),
            pltpu.VMEM((2 * 16, 128), jnp.float32),
            pltpu.SemaphoreType.DMA,
            pltpu.SemaphoreType.DMA,
            pltpu.SemaphoreType.DMA,
        ],
        name="sc_decode",
    )
    def dec_kernel(emb_hbm, a0_hbm, a1_hbm, b0_hbm, b1_hbm, d0_hbm, d1_hbm,
                   av0, bv0, av1, bv1, bufa, bufb, outv, sema, semb, semo):
        cid = lax.axis_index("c")
        sid = lax.axis_index("s")

        def work(avx, bvx, nblx, dx_hbm):
            obase = sid * (nblx * 16)

            @pl.loop(0, nblx)
            def _(j):
                ca = pltpu.async_copy(emb_hbm.at[avx.at[j]], bufa, sema)
                cb = pltpu.async_copy(emb_hbm.at[bvx.at[j]], bufb, semb)
                ca.wait()
                cb.wait()
                par16 = lax.rem(j, 2) * 16

                @pl.when(j >= 2)
                def _():
                    pltpu.make_async_copy(
                        outv.at[pl.ds(par16, 16)],
                        dx_hbm.at[pl.ds(obase, 16)], semo,
                    ).wait()

                @pl.loop(0, 128)
                def _(i):
                    acc = (
                        bufa[pl.ds(i, 1), pl.ds(0, L)]
                        * bufb[pl.ds(i, 1), pl.ds(0, L)]
                    )
                    for c in range(1, D // L):
                        acc = acc + (
                            bufa[pl.ds(i, 1), pl.ds(c * L, L)]
                            * bufb[pl.ds(i, 1), pl.ds(c * L, L)]
                        )
                    outv[pl.ds(par16 + i // 8, 1), pl.ds((i % 8) * L, L)] = acc

                pltpu.async_copy(
                    outv.at[pl.ds(par16, 16)],
                    dx_hbm.at[pl.ds(obase + j * 16, 16)], semo,
                )

            @pl.loop(0, 2)
            def _(k):
                pltpu.make_async_copy(
                    outv.at[pl.ds(0, 16)], dx_hbm.at[pl.ds(obase, 16)], semo
                ).wait()

        @pl.when(cid == 0)
        def _():
            pltpu.sync_copy(a0_hbm.at[sid], av0)
            pltpu.sync_copy(b0_hbm.at[sid], bv0)
            work(av0, bv0, NBL0, d0_hbm)

        @pl.when(cid == 1)
        def _():
            pltpu.sync_copy(a1_hbm.at[sid], av1)
            pltpu.sync_copy(b1_hbm.at[sid], bv1)
            work(av1, bv1, NBL1, d1_hbm)

    return dec_kernel(agg2, a0, a1, b0, b1)


_DGRID = 4
_DROWS = ELPAD // 128 // _DGRID


def _decode_finish_call(dots3):
    def body(d_ref, o_ref):
        o_ref[...] = jax.nn.sigmoid(jnp.sum(d_ref[...], axis=-1))

    return pl.pallas_call(
        body,
        grid=(_DGRID,),
        in_specs=[pl.BlockSpec((_DROWS, 128, L), lambda i: (i, 0, 0))],
        out_specs=pl.BlockSpec((_DROWS, 128), lambda i: (i, 0)),
        out_shape=jax.ShapeDtypeStruct((ELPAD // 128, 128), jnp.float32),
    )(dots3)


_GRID = 4
_BLK = NPAD // _GRID


def _row_spec(w):
    return pl.BlockSpec((_BLK, w), lambda i: (i, 0))


def _full_spec(h, w):
    return pl.BlockSpec((h, w), lambda i: (0, 0))


def _dinv(dga_ref, dgb_ref):
    deg = dga_ref[:, :1] + dgb_ref[:, :1] + 1.0
    return lax.rsqrt(deg)


def _xp_call(dga, dgb, x_pad):
    def body(dga_ref, dgb_ref, x_ref, o_ref):
        o_ref[...] = x_ref[...] * _dinv(dga_ref, dgb_ref)

    return pl.pallas_call(
        body,
        grid=(_GRID,),
        in_specs=[_row_spec(L), _row_spec(L), _row_spec(D)],
        out_specs=_row_spec(D),
        out_shape=jax.ShapeDtypeStruct((NPAD, D), jnp.float32),
    )(dga, dgb, x_pad)


def _layer1_call(p0, p1, xp, dga, dgb, W1, b1):
    def body(p0_ref, p1_ref, xp_ref, dga_ref, dgb_ref, w_ref, b_ref, o_ref):
        dinv = _dinv(dga_ref, dgb_ref)
        agg = (p0_ref[...] + p1_ref[...] + xp_ref[...]) * dinv
        h = jnp.dot(agg, w_ref[...], precision=lax.Precision.HIGHEST) + b_ref[...]
        o_ref[...] = jnp.maximum(h, 0.0) * dinv

    return pl.pallas_call(
        body,
        grid=(_GRID,),
        in_specs=[
            _row_spec(D), _row_spec(D), _row_spec(D),
            _row_spec(L), _row_spec(L),
            _full_spec(D, D), _full_spec(1, D),
        ],
        out_specs=_row_spec(D),
        out_shape=jax.ShapeDtypeStruct((NPAD, D), jnp.float32),
    )(p0, p1, xp, dga, dgb, W1, b1)


def _layer2_call(q0, q1, h1p, dga, dgb, W2, b2):
    def body(q0_ref, q1_ref, h1p_ref, dga_ref, dgb_ref, w_ref, b_ref,
             agg_ref, h_ref):
        dinv = _dinv(dga_ref, dgb_ref)
        agg = (q0_ref[...] + q1_ref[...] + h1p_ref[...]) * dinv
        agg_ref[...] = agg
        h_ref[...] = (
            jnp.dot(agg, w_ref[...], precision=lax.Precision.HIGHEST) + b_ref[...]
        )

    return pl.pallas_call(
        body,
        grid=(_GRID,),
        in_specs=[
            _row_spec(D), _row_spec(D), _row_spec(D),
            _row_spec(L), _row_spec(L),
            _full_spec(D, D), _full_spec(1, D),
        ],
        out_specs=[_row_spec(D), _row_spec(D)],
        out_shape=[
            jax.ShapeDtypeStruct((NPAD, D), jnp.float32),
            jax.ShapeDtypeStruct((NPAD, D), jnp.float32),
        ],
    )(q0, q1, h1p, dga, dgb, W2, b2)


def _pad_idx(idx, total, fill_base, fill_mod):
    npad = total - idx.shape[0]
    pad = fill_base + (jnp.arange(npad, dtype=jnp.int32) % fill_mod)
    return jnp.concatenate([idx.astype(jnp.int32), pad])


def kernel(x, edge_index, edge_label_index, W1, b1, W2, b2):
    src2 = _pad_idx(edge_index[0], EPAD, DUMMY, NPAD - N).reshape(TOTB, EB)
    dst2 = _pad_idx(edge_index[1], EPAD, DUMMY, NPAD - N).reshape(TOTB, EB)
    a2 = _pad_idx(edge_label_index[0], ELPAD, 0, N).reshape(TOTBL, 128)
    b2_idx = _pad_idx(edge_label_index[1], ELPAD, 0, N).reshape(TOTBL, 128)
    split = NS * NBL0
    ai0 = a2[:split].reshape(NS, NBL0, 128)
    ai1 = a2[split:].reshape(NS, NBL1, 128)
    bi0 = b2_idx[:split].reshape(NS, NBL0, 128)
    bi1 = b2_idx[split:].reshape(NS, NBL1, 128)

    x_pad = jnp.pad(x, ((0, NPAD - N), (0, 0)))
    z16 = jnp.zeros((NPAD, L), jnp.float32)
    z128 = jnp.zeros((NPAD, D), jnp.float32)

    degp = _deg_call(dst2, z16)
    dga, dgb = degp[0], degp[1]

    xp = _xp_call(dga, dgb, x_pad)
    p = _prop_call(xp, src2, dst2, z128)
    h1p = _layer1_call(p[0], p[1], xp, dga, dgb, W1, b1.reshape(1, D))
    q = _prop_call(h1p, src2, dst2, z128)
    agg2, h2 = _layer2_call(q[0], q[1], h1p, dga, dgb, W2, b2.reshape(1, D))
    d0, d1 = _decode_call(agg2, ai0, ai1, bi0, bi1)
    dots = jnp.concatenate([d0, d1], axis=0)
    r = _decode_finish_call(dots.reshape(ELPAD // 128, 128, L)).reshape(ELPAD)

    return (h2[:N], r[:EL])


# speedup vs baseline: 3.9051x; 1.2970x over previous
import functools

import jax
import jax.numpy as jnp
from jax import lax
from jax.experimental import pallas as pl
from jax.experimental.pallas import tpu as pltpu
from jax.experimental.pallas import tpu_sc as plsc

N = 10000
E = 320000
EL = 100000
D = 128

NC = 2
NS = 16
NW = NC * NS
L = 16

NPAD = 10240
RPS = NPAD // NS
DUMMY = N

EB = 128
TOTB = 2560
EPAD = TOTB * EB
NB0 = 80
NB1 = 80
CHB = 8
NCH0 = NB0 // CHB
NCH1 = NB1 // CHB
NB_DEG = TOTB // NW

TOTBL = 896
ELPAD = TOTBL * 128
NBL0 = 32
NBL1 = 24


def _mesh():
    return plsc.VectorSubcoreMesh(
        core_axis_name="c", subcore_axis_name="s", num_cores=NC, num_subcores=NS
    )


def _deg_call(dst2, z16):
    @functools.partial(
        pl.kernel,
        out_type=jax.ShapeDtypeStruct((NC, NPAD, L), jnp.float32),
        mesh=_mesh(),
        scratch_types=[
            pltpu.VMEM((NB_DEG, EB), jnp.int32),
            pltpu.VMEM((EB, L), jnp.float32),
            pltpu.VMEM_SHARED((NPAD, L), jnp.float32),
            pltpu.SemaphoreType.DMA,
        ],
        name="sc_deg",
    )
    def deg_kernel(dst_hbm, z_hbm, out_hbm, dstv, onesv, acc, semz):
        cid = lax.axis_index("c")
        sid = lax.axis_index("s")
        wid = sid * NC + cid

        zc = pltpu.async_copy(
            z_hbm.at[pl.ds(sid * RPS, RPS)], acc.at[pl.ds(sid * RPS, RPS)], semz
        )
        pltpu.sync_copy(dst_hbm.at[pl.ds(wid * NB_DEG, NB_DEG)], dstv)

        @pl.loop(0, EB)
        def _(i):
            onesv[pl.ds(i, 1), :] = jnp.ones((1, L), jnp.float32)

        zc.wait()
        plsc.subcore_barrier()

        @pl.loop(0, NB_DEG)
        def _(j):
            pltpu.sync_copy(onesv, acc.at[dstv.at[j]], add=True)

        plsc.subcore_barrier()
        pltpu.sync_copy(
            acc.at[pl.ds(sid * RPS, RPS)], out_hbm.at[cid].at[pl.ds(sid * RPS, RPS)]
        )

    return deg_kernel(dst2, z16)


def _prop_call(xp, src2, dst2, z128):
    @functools.partial(
        pl.kernel,
        out_type=jax.ShapeDtypeStruct((NC, NPAD, D), jnp.float32),
        mesh=_mesh(),
        scratch_types=[
            pltpu.VMEM((2 * CHB, EB), jnp.int32),
            pltpu.VMEM((2 * CHB, EB), jnp.int32),
            pltpu.VMEM((EB, D), jnp.float32),
            pltpu.VMEM((EB, D), jnp.float32),
            pltpu.VMEM_SHARED((NPAD, D), jnp.float32),
            pltpu.SemaphoreType.DMA,
            pltpu.SemaphoreType.DMA,
            pltpu.SemaphoreType.DMA,
            pltpu.SemaphoreType.DMA,
            pltpu.SemaphoreType.DMA,
        ],
        name="sc_prop",
    )
    def prop_kernel(
        xp_hbm, src_hbm, dst_hbm, z_hbm, out_hbm,
        srcv, dstv, bufa, bufb, acc, semz, sema, semb, semis, semid,
    ):
        cid = lax.axis_index("c")
        sid = lax.axis_index("s")

        nch = jnp.where(cid == 0, NCH0, NCH1)
        wb = jnp.where(cid == 0, sid * NB0, NS * NB0 + sid * NB1)

        zc = pltpu.async_copy(
            z_hbm.at[pl.ds(sid * RPS, RPS)], acc.at[pl.ds(sid * RPS, RPS)], semz
        )
        pltpu.sync_copy(src_hbm.at[pl.ds(wb, CHB)], srcv.at[pl.ds(0, CHB)])
        pltpu.sync_copy(dst_hbm.at[pl.ds(wb, CHB)], dstv.at[pl.ds(0, CHB)])
        zc.wait()
        plsc.subcore_barrier()

        pltpu.async_copy(xp_hbm.at[srcv.at[0]], bufa, sema)

        @pl.loop(0, nch)
        def _(c):
            par = lax.rem(c, 2)
            base = par * CHB
            nbase = (1 - par) * CHB

            @pl.when(c + 1 < nch)
            def _():
                pltpu.async_copy(
                    src_hbm.at[pl.ds(wb + (c + 1) * CHB, CHB)],
                    srcv.at[pl.ds(nbase, CHB)], semis,
                )
                pltpu.async_copy(
                    dst_hbm.at[pl.ds(wb + (c + 1) * CHB, CHB)],
                    dstv.at[pl.ds(nbase, CHB)], semid,
                )

            @pl.loop(0, CHB, step=2)
            def _(jj):
                pltpu.make_async_copy(
                    xp_hbm.at[srcv.at[base + jj]], bufa, sema
                ).wait()
                pltpu.async_copy(xp_hbm.at[srcv.at[base + jj + 1]], bufb, semb)
                pltpu.sync_copy(bufa, acc.at[dstv.at[base + jj]], add=True)
                pltpu.make_async_copy(
                    xp_hbm.at[srcv.at[base + jj + 1]], bufb, semb
                ).wait()

                @pl.when(jj + 2 < CHB)
                def _():
                    pltpu.async_copy(
                        xp_hbm.at[srcv.at[base + jj + 2]], bufa, sema
                    )

                pltpu.sync_copy(bufb, acc.at[dstv.at[base + jj + 1]], add=True)

            @pl.when(c + 1 < nch)
            def _():
                pltpu.make_async_copy(
                    src_hbm.at[pl.ds(wb + (c + 1) * CHB, CHB)],
                    srcv.at[pl.ds(nbase, CHB)], semis,
                ).wait()
                pltpu.make_async_copy(
                    dst_hbm.at[pl.ds(wb + (c + 1) * CHB, CHB)],
                    dstv.at[pl.ds(nbase, CHB)], semid,
                ).wait()
                pltpu.async_copy(xp_hbm.at[srcv.at[nbase]], bufa, sema)

        plsc.subcore_barrier()
        pltpu.sync_copy(
            acc.at[pl.ds(sid * RPS, RPS)], out_hbm.at[cid].at[pl.ds(sid * RPS, RPS)]
        )

    return prop_kernel(xp, src2, dst2, z128)


def _decode_call(agg2, a0, a1, b0, b1):
    @functools.partial(
        pl.kernel,
        out_type=[
            jax.ShapeDtypeStruct((NS * NBL0 * 16, 128), jnp.float32),
            jax.ShapeDtypeStruct((NS * NBL1 * 16, 128), jnp.float32),
        ],
        mesh=_mesh(),
        scratch_types=[
            pltpu.VMEM((NBL0, 128), jnp.int32),
            pltpu.VMEM((NBL0, 128), jnp.int32),
            pltpu.VMEM((NBL1, 128), jnp.int32),
            pltpu.VMEM((NBL1, 128), jnp.int32),
            pltpu.VMEM((128, D), jnp.float32),
            pltpu.VMEM((128, D), jnp.float32),
            pltpu.VMEM((2 * 16, 128), jnp.float32),
            pltpu.SemaphoreType.DMA,
            pltpu.SemaphoreType.DMA,
            pltpu.SemaphoreType.DMA,
        ],
        name="sc_decode",
    )
    def dec_kernel(emb_hbm, a0_hbm, a1_hbm, b0_hbm, b1_hbm, d0_hbm, d1_hbm,
                   av0, bv0, av1, bv1, bufa, bufb, outv, sema, semb, semo):
        cid = lax.axis_index("c")
        sid = lax.axis_index("s")

        def work(avx, bvx, nblx, dx_hbm):
            obase = sid * (nblx * 16)

            @pl.loop(0, nblx)
            def _(j):
                ca = pltpu.async_copy(emb_hbm.at[avx.at[j]], bufa, sema)
                cb = pltpu.async_copy(emb_hbm.at[bvx.at[j]], bufb, semb)
                ca.wait()
                cb.wait()
                par16 = lax.rem(j, 2) * 16

                @pl.when(j >= 2)
                def _():
                    pltpu.make_async_copy(
                        outv.at[pl.ds(par16, 16)],
                        dx_hbm.at[pl.ds(obase, 16)], semo,
                    ).wait()

                @pl.loop(0, 128)
                def _(i):
                    acc = (
                        bufa[pl.ds(i, 1), pl.ds(0, L)]
                        * bufb[pl.ds(i, 1), pl.ds(0, L)]
                    )
                    for c in range(1, D // L):
                        acc = acc + (
                            bufa[pl.ds(i, 1), pl.ds(c * L, L)]
                            * bufb[pl.ds(i, 1), pl.ds(c * L, L)]
                        )
                    outv[pl.ds(par16 + i // 8, 1), pl.ds((i % 8) * L, L)] = acc

                pltpu.async_copy(
                    outv.at[pl.ds(par16, 16)],
                    dx_hbm.at[pl.ds(obase + j * 16, 16)], semo,
                )

            @pl.loop(0, 2)
            def _(k):
                pltpu.make_async_copy(
                    outv.at[pl.ds(0, 16)], dx_hbm.at[pl.ds(obase, 16)], semo
                ).wait()

        @pl.when(cid == 0)
        def _():
            pltpu.sync_copy(a0_hbm.at[sid], av0)
            pltpu.sync_copy(b0_hbm.at[sid], bv0)
            work(av0, bv0, NBL0, d0_hbm)

        @pl.when(cid == 1)
        def _():
            pltpu.sync_copy(a1_hbm.at[sid], av1)
            pltpu.sync_copy(b1_hbm.at[sid], bv1)
            work(av1, bv1, NBL1, d1_hbm)

    return dec_kernel(agg2, a0, a1, b0, b1)


_DGRID = 4
_DROWS = ELPAD // 128 // _DGRID


def _decode_finish_call(dots3):
    def body(d_ref, o_ref):
        o_ref[...] = jax.nn.sigmoid(jnp.sum(d_ref[...], axis=-1))

    return pl.pallas_call(
        body,
        grid=(_DGRID,),
        in_specs=[pl.BlockSpec((_DROWS, 128, L), lambda i: (i, 0, 0))],
        out_specs=pl.BlockSpec((_DROWS, 128), lambda i: (i, 0)),
        out_shape=jax.ShapeDtypeStruct((ELPAD // 128, 128), jnp.float32),
    )(dots3)


_GRID = 4
_BLK = NPAD // _GRID


def _row_spec(w):
    return pl.BlockSpec((_BLK, w), lambda i: (i, 0))


def _full_spec(h, w):
    return pl.BlockSpec((h, w), lambda i: (0, 0))


def _dinv(dga_ref, dgb_ref):
    deg = dga_ref[:, :1] + dgb_ref[:, :1] + 1.0
    return lax.rsqrt(deg)


def _xp_call(dga, dgb, x_pad):
    def body(dga_ref, dgb_ref, x_ref, o_ref):
        o_ref[...] = x_ref[...] * _dinv(dga_ref, dgb_ref)

    return pl.pallas_call(
        body,
        grid=(_GRID,),
        in_specs=[_row_spec(L), _row_spec(L), _row_spec(D)],
        out_specs=_row_spec(D),
        out_shape=jax.ShapeDtypeStruct((NPAD, D), jnp.float32),
    )(dga, dgb, x_pad)


def _layer1_call(p0, p1, xp, dga, dgb, W1, b1):
    def body(p0_ref, p1_ref, xp_ref, dga_ref, dgb_ref, w_ref, b_ref, o_ref):
        dinv = _dinv(dga_ref, dgb_ref)
        agg = (p0_ref[...] + p1_ref[...] + xp_ref[...]) * dinv
        h = jnp.dot(agg, w_ref[...], precision=lax.Precision.HIGHEST) + b_ref[...]
        o_ref[...] = jnp.maximum(h, 0.0) * dinv

    return pl.pallas_call(
        body,
        grid=(_GRID,),
        in_specs=[
            _row_spec(D), _row_spec(D), _row_spec(D),
            _row_spec(L), _row_spec(L),
            _full_spec(D, D), _full_spec(1, D),
        ],
        out_specs=_row_spec(D),
        out_shape=jax.ShapeDtypeStruct((NPAD, D), jnp.float32),
    )(p0, p1, xp, dga, dgb, W1, b1)


def _layer2_call(q0, q1, h1p, dga, dgb, W2, b2):
    def body(q0_ref, q1_ref, h1p_ref, dga_ref, dgb_ref, w_ref, b_ref,
             agg_ref, h_ref):
        dinv = _dinv(dga_ref, dgb_ref)
        agg = (q0_ref[...] + q1_ref[...] + h1p_ref[...]) * dinv
        agg_ref[...] = agg
        h_ref[...] = (
            jnp.dot(agg, w_ref[...], precision=lax.Precision.HIGHEST) + b_ref[...]
        )

    return pl.pallas_call(
        body,
        grid=(_GRID,),
        in_specs=[
            _row_spec(D), _row_spec(D), _row_spec(D),
            _row_spec(L), _row_spec(L),
            _full_spec(D, D), _full_spec(1, D),
        ],
        out_specs=[_row_spec(D), _row_spec(D)],
        out_shape=[
            jax.ShapeDtypeStruct((NPAD, D), jnp.float32),
            jax.ShapeDtypeStruct((NPAD, D), jnp.float32),
        ],
    )(q0, q1, h1p, dga, dgb, W2, b2)


def _pad_idx(idx, total, fill_base, fill_mod):
    npad = total - idx.shape[0]
    pad = fill_base + (jnp.arange(npad, dtype=jnp.int32) % fill_mod)
    return jnp.concatenate([idx.astype(jnp.int32), pad])


def kernel(x, edge_index, edge_label_index, W1, b1, W2, b2):
    src2 = _pad_idx(edge_index[0], EPAD, DUMMY, NPAD - N).reshape(TOTB, EB)
    dst2 = _pad_idx(edge_index[1], EPAD, DUMMY, NPAD - N).reshape(TOTB, EB)
    a2 = _pad_idx(edge_label_index[0], ELPAD, 0, N).reshape(TOTBL, 128)
    b2_idx = _pad_idx(edge_label_index[1], ELPAD, 0, N).reshape(TOTBL, 128)
    split = NS * NBL0
    ai0 = a2[:split].reshape(NS, NBL0, 128)
    ai1 = a2[split:].reshape(NS, NBL1, 128)
    bi0 = b2_idx[:split].reshape(NS, NBL0, 128)
    bi1 = b2_idx[split:].reshape(NS, NBL1, 128)

    x_pad = jnp.pad(x, ((0, NPAD - N), (0, 0)))
    z16 = jnp.zeros((NPAD, L), jnp.float32)
    z128 = jnp.zeros((NPAD, D), jnp.float32)

    degp = _deg_call(dst2, z16)
    dga, dgb = degp[0], degp[1]

    xp = _xp_call(dga, dgb, x_pad)
    p = _prop_call(xp, src2, dst2, z128)
    h1p = _layer1_call(p[0], p[1], xp, dga, dgb, W1, b1.reshape(1, D))
    q = _prop_call(h1p, src2, dst2, z128)
    agg2, h2 = _layer2_call(q[0], q[1], h1p, dga, dgb, W2, b2.reshape(1, D))
    d0, d1 = _decode_call(agg2, ai0, ai1, bi0, bi1)
    dots = jnp.concatenate([d0, d1], axis=0)
    r = _decode_finish_call(dots.reshape(ELPAD // 128, 128, L)).reshape(ELPAD)

    return (h2[:N], r[:EL])
